# baseline (speedup 1.0000x reference)
import numpy as np

import jax
import jax.numpy as jnp
from jax.experimental import pallas as pl
from jax.experimental.pallas import tpu as pltpu

_LN_EPS = 1e-5
_MED_EPS = 1e-6
_SQRT_2_OVER_PI = 0.7978845608028654


def _gelu(x):
    return 0.5 * x * (1.0 + jnp.tanh(_SQRT_2_OVER_PI * (x + 0.044715 * x * x * x)))


def _idft_half_mats(N, F, Fp):
    nh = N // 2 + 1
    nhp = ((nh + 7) // 8) * 8
    n = np.arange(nh)[None, :]
    f = np.arange(F)[:, None]
    ang = 2.0 * np.pi * f * n / N
    scale = 1.0 / np.sqrt(N)
    w = np.full((F,), 2.0)
    w[0] = 1.0
    if N % 2 == 0:
        w[-1] = 1.0
    ar = np.zeros((nhp, Fp), np.float64)
    ai = np.zeros((nhp, Fp), np.float64)
    ar[:nh, :F] = np.cos(ang).T * (w * scale)
    ai[:nh, :F] = -np.sin(ang).T * (w * scale)
    return ar, ai


def _pass_a(x_ref, g_ref, s_ref, ln1w_ref, ln1b_ref, fst_ref, xfft_ref, energy_ref):
    x = x_ref[0]
    mu = jnp.dot(x, g_ref[...], preferred_element_type=jnp.float32)
    xc = x - mu
    var = jnp.dot(xc * xc, g_ref[...], preferred_element_type=jnp.float32)
    xn = xc * jax.lax.rsqrt(var + _LN_EPS) * ln1w_ref[...] + ln1b_ref[...]
    X = jnp.dot(fst_ref[...], xn.astype(fst_ref.dtype),
                preferred_element_type=jnp.float32)
    xfft_ref[0] = X.astype(xfft_ref.dtype)
    p = X * X
    e2 = jnp.dot(p, s_ref[...], preferred_element_type=jnp.float32)
    fp = e2.shape[0] // 2
    energy_ref[0] = e2[:fp] + e2[fp:]


def _pass_b(xres_ref, xfft_ref, mask_ref, cwr_ref, cwi_ref, chr_ref, chi_ref,
            arh_ref, aih_ref, rev_ref, ln2w_ref, ln2b_ref,
            w1_ref, b1_ref, w2_ref, b2_ref, w3_ref, b3_ref, o_ref):
    for j in range(xfft_ref.shape[0]):
        X = xfft_ref[j].astype(jnp.float32)
        fp = X.shape[0] // 2
        Xr, Xi = X[:fp], X[fp:]

        mask = mask_ref[j]
        wr_eff = cwr_ref[...] + mask * chr_ref[...]
        wi_eff = cwi_ref[...] + mask * chi_ref[...]
        Wr = Xr * wr_eff - Xi * wi_eff
        Wi = Xr * wi_eff + Xi * wr_eff

        mmt = arh_ref.dtype
        u = jnp.dot(arh_ref[...], Wr.astype(mmt), preferred_element_type=jnp.float32)
        v = jnp.dot(aih_ref[...], Wi.astype(mmt), preferred_element_type=jnp.float32)
        n = xres_ref.shape[1]
        nh = n // 2
        top = u[:nh] + v[:nh]
        d = (u[1:nh + 1] - v[1:nh + 1]).astype(rev_ref.dtype)
        bot = jnp.dot(rev_ref[...], d, preferred_element_type=jnp.float32)
        xa = jnp.concatenate([top, bot], axis=0)

        mu = jnp.mean(xa, axis=1, keepdims=True)
        xc = xa - mu
        var = jnp.mean(xc * xc, axis=1, keepdims=True)
        y = xc * jax.lax.rsqrt(var + _LN_EPS) * ln2w_ref[...] + ln2b_ref[...]

        mm = w1_ref.dtype
        y_m = y.astype(mm)
        z = jnp.zeros((1, y.shape[1]), mm)
        y_prev = jnp.concatenate([z, y_m[:n - 1]], axis=0)
        y_next = jnp.concatenate([y_m[1:], z], axis=0)

        x1 = jnp.dot(y_m, w1_ref[...], preferred_element_type=jnp.float32) + b1_ref[...]
        taps = jnp.concatenate([y_prev, y_m, y_next], axis=1)
        x2 = jnp.dot(taps, w2_ref[...], preferred_element_type=jnp.float32) + b2_ref[...]
        g1 = _gelu(x1)
        g2 = _gelu(x2)
        out = x1 * g2 + x2 * g1
        icb = jnp.dot(out.astype(w3_ref.dtype), w3_ref[...],
                      preferred_element_type=jnp.float32) + b3_ref[...]
        o_ref[j] = xres_ref[j] + icb


@jax.jit
def kernel(x, cw, cwh, threshold, ln1_w, ln1_b, ln2_w, ln2_b,
           conv1_w, conv1_b, conv2_w, conv2_b, conv3_w, conv3_b):
    orig_dtype = x.dtype
    B, N, C = x.shape
    F = N // 2 + 1
    Fp = ((F + 7) // 8) * 8
    H = conv1_w.shape[1]
    f32 = jnp.float32
    mm = jnp.bfloat16

    d = jnp.fft.rfft(jnp.eye(N, dtype=f32), axis=0, norm="ortho")
    fr = jnp.zeros((Fp, N), f32).at[:F].set(jnp.real(d))
    fi = jnp.zeros((Fp, N), f32).at[:F].set(jnp.imag(d))
    fstack = jnp.concatenate([fr, fi], axis=0).astype(mm)
    arh_np, aih_np = _idft_half_mats(N, F, Fp)
    arh = jnp.asarray(arh_np.astype(np.float32), dtype=mm)
    aih = jnp.asarray(aih_np.astype(np.float32), dtype=mm)
    NHP = arh_np.shape[0]
    revm = jnp.asarray(np.eye(N // 2, dtype=np.float32)[::-1], dtype=mm)

    gmat = jnp.full((C, C), 1.0 / C, f32)
    smat = jnp.ones((C, 1), f32)
    xf = x.astype(f32)
    row = lambda v: v.astype(f32).reshape(1, -1)
    ln1w2, ln1b2 = row(ln1_w), row(ln1_b)
    ln2w2, ln2b2 = row(ln2_w), row(ln2_b)
    cwr, cwi = row(cw[:, 0]), row(cw[:, 1])
    chr_, chi_ = row(cwh[:, 0]), row(cwh[:, 1])
    w1 = conv1_w.astype(f32).astype(mm)
    w2 = conv2_w.astype(f32).reshape(3 * C, H).astype(mm)
    w3 = conv3_w.astype(f32).astype(mm)
    b1, b2 = row(conv1_b), row(conv2_b)
    b3 = row(conv3_b)

    cparams = pltpu.CompilerParams(
        dimension_semantics=("parallel",),
        vmem_limit_bytes=64 * 1024 * 1024,
    )
    bat = lambda i: (i, 0, 0)
    c2 = lambda i: (0, 0)
    BT = 2

    xfft, energy3 = pl.pallas_call(
        _pass_a,
        grid=(B,),
        in_specs=[
            pl.BlockSpec((1, N, C), bat),
            pl.BlockSpec((C, C), c2),
            pl.BlockSpec((C, 1), c2),
            pl.BlockSpec((1, C), c2),
            pl.BlockSpec((1, C), c2),
            pl.BlockSpec((2 * Fp, N), c2),
        ],
        out_specs=(
            pl.BlockSpec((1, 2 * Fp, C), bat),
            pl.BlockSpec((1, Fp, 1), bat),
        ),
        out_shape=(
            jax.ShapeDtypeStruct((B, 2 * Fp, C), mm),
            jax.ShapeDtypeStruct((B, Fp, 1), f32),
        ),
        compiler_params=cparams,
    )(xf, gmat, smat, ln1w2, ln1b2, fstack)

    pad = Fp - F
    energy = energy3.reshape(B, Fp)
    med = jnp.sort(energy, axis=1)[:, (F - 1) // 2 + pad][:, None]
    ne = energy / (med + _MED_EPS)
    nf = B * F
    v = jnp.sort(ne.ravel())
    pos = threshold.reshape(()).astype(f32) * np.float32(nf - 1)
    low = jnp.clip(jnp.floor(pos), 0, nf - 1).astype(jnp.int32)
    thr = jax.lax.dynamic_index_in_dim(v, B * pad + low, keepdims=False)
    mask3 = (ne > thr).astype(f32).reshape(B, Fp, 1)

    out = pl.pallas_call(
        _pass_b,
        grid=(B // BT,),
        in_specs=[
            pl.BlockSpec((BT, N, C), bat),
            pl.BlockSpec((BT, 2 * Fp, C), bat),
            pl.BlockSpec((BT, Fp, 1), bat),
            pl.BlockSpec((1, C), c2),
            pl.BlockSpec((1, C), c2),
            pl.BlockSpec((1, C), c2),
            pl.BlockSpec((1, C), c2),
            pl.BlockSpec((NHP, Fp), c2),
            pl.BlockSpec((NHP, Fp), c2),
            pl.BlockSpec((N // 2, N // 2), c2),
            pl.BlockSpec((1, C), c2),
            pl.BlockSpec((1, C), c2),
            pl.BlockSpec((C, H), c2),
            pl.BlockSpec((1, H), c2),
            pl.BlockSpec((3 * C, H), c2),
            pl.BlockSpec((1, H), c2),
            pl.BlockSpec((H, C), c2),
            pl.BlockSpec((1, C), c2),
        ],
        out_specs=pl.BlockSpec((BT, N, C), bat),
        out_shape=jax.ShapeDtypeStruct((B, N, C), f32),
        compiler_params=cparams,
    )(xf, xfft, mask3, cwr, cwi, chr_, chi_,
      arh, aih, revm, ln2w2, ln2b2, w1, b1, w2, b2, w3, b3)

    return out.astype(orig_dtype)


# speedup vs baseline: 1.0751x; 1.0751x over previous
import numpy as np

import jax
import jax.numpy as jnp
from jax.experimental import pallas as pl
from jax.experimental.pallas import tpu as pltpu

_LN_EPS = 1e-5
_MED_EPS = 1e-6
_SQRT_2_OVER_PI = 0.7978845608028654


def _gelu(x):
    return 0.5 * x * (1.0 + jnp.tanh(_SQRT_2_OVER_PI * (x + 0.044715 * x * x * x)))


def _idft_half_mats(N, F, Fp):
    nh = N // 2 + 1
    nhp = ((nh + 7) // 8) * 8
    n = np.arange(nh)[None, :]
    f = np.arange(F)[:, None]
    ang = 2.0 * np.pi * f * n / N
    scale = 1.0 / np.sqrt(N)
    w = np.full((F,), 2.0)
    w[0] = 1.0
    if N % 2 == 0:
        w[-1] = 1.0
    ar = np.zeros((nhp, Fp), np.float64)
    ai = np.zeros((nhp, Fp), np.float64)
    ar[:nh, :F] = np.cos(ang).T * (w * scale)
    ai[:nh, :F] = -np.sin(ang).T * (w * scale)
    return ar, ai


def _pass_a(x_ref, g_ref, s_ref, ln1w_ref, ln1b_ref, fst_ref, xfft_ref, energy_ref):
    x = x_ref[0]
    mu = jnp.dot(x, g_ref[...], preferred_element_type=jnp.float32)
    xc = x - mu
    var = jnp.dot(xc * xc, g_ref[...], preferred_element_type=jnp.float32)
    xn = xc * jax.lax.rsqrt(var + _LN_EPS) * ln1w_ref[...] + ln1b_ref[...]
    X = jnp.dot(fst_ref[...], xn.astype(fst_ref.dtype),
                preferred_element_type=jnp.float32)
    xfft_ref[0] = X.astype(xfft_ref.dtype)
    p = X * X
    e2 = jnp.dot(p, s_ref[...], preferred_element_type=jnp.float32)
    fp = e2.shape[0] // 2
    energy_ref[0] = e2[:fp] + e2[fp:]


def _pass_b(xres_ref, xfft_ref, mask_ref, cwr_ref, cwi_ref, chr_ref, chi_ref,
            arh_ref, aih_ref, rev_ref, ln2w_ref, ln2b_ref,
            w1_ref, b1_ref, w2_ref, b2_ref, w3_ref, b3_ref, o_ref):
    for j in range(xfft_ref.shape[0]):
        X = xfft_ref[j].astype(jnp.float32)
        fp = X.shape[0] // 2
        Xr, Xi = X[:fp], X[fp:]

        mask = mask_ref[j]
        wr_eff = cwr_ref[...] + mask * chr_ref[...]
        wi_eff = cwi_ref[...] + mask * chi_ref[...]
        Wr = Xr * wr_eff - Xi * wi_eff
        Wi = Xr * wi_eff + Xi * wr_eff

        mmt = arh_ref.dtype
        u = jnp.dot(arh_ref[...], Wr.astype(mmt), preferred_element_type=jnp.float32)
        v = jnp.dot(aih_ref[...], Wi.astype(mmt), preferred_element_type=jnp.float32)
        n = xres_ref.shape[1]
        nh = n // 2
        top = u[:nh] + v[:nh]
        d = (u[1:nh + 1] - v[1:nh + 1]).astype(rev_ref.dtype)
        bot = jnp.dot(rev_ref[...], d, preferred_element_type=jnp.float32)
        xa = jnp.concatenate([top, bot], axis=0)

        mu = jnp.mean(xa, axis=1, keepdims=True)
        xc = xa - mu
        var = jnp.mean(xc * xc, axis=1, keepdims=True)
        y = xc * jax.lax.rsqrt(var + _LN_EPS) * ln2w_ref[...] + ln2b_ref[...]

        mm = w1_ref.dtype
        y_m = y.astype(mm)
        z = jnp.zeros((1, y.shape[1]), mm)
        y_prev = jnp.concatenate([z, y_m[:n - 1]], axis=0)
        y_next = jnp.concatenate([y_m[1:], z], axis=0)

        x1 = jnp.dot(y_m, w1_ref[...], preferred_element_type=jnp.float32) + b1_ref[...]
        taps = jnp.concatenate([y_prev, y_m, y_next], axis=1)
        x2 = jnp.dot(taps, w2_ref[...], preferred_element_type=jnp.float32) + b2_ref[...]
        xb1 = x1.astype(mm)
        xb2 = x2.astype(mm)
        a = jnp.asarray(0.044715, mm)
        c = jnp.asarray(_SQRT_2_OVER_PI, mm)
        t1 = jnp.tanh(c * (xb1 + a * xb1 * xb1 * xb1))
        t2 = jnp.tanh(c * (xb2 + a * xb2 * xb2 * xb2))
        s = jnp.asarray(1.0, mm) + jnp.asarray(0.5, mm) * (t1 + t2)
        out = xb1 * xb2 * s
        icb = jnp.dot(out, w3_ref[...],
                      preferred_element_type=jnp.float32) + b3_ref[...]
        o_ref[j] = xres_ref[j] + icb


@jax.jit
def kernel(x, cw, cwh, threshold, ln1_w, ln1_b, ln2_w, ln2_b,
           conv1_w, conv1_b, conv2_w, conv2_b, conv3_w, conv3_b):
    orig_dtype = x.dtype
    B, N, C = x.shape
    F = N // 2 + 1
    Fp = ((F + 7) // 8) * 8
    H = conv1_w.shape[1]
    f32 = jnp.float32
    mm = jnp.bfloat16

    d = jnp.fft.rfft(jnp.eye(N, dtype=f32), axis=0, norm="ortho")
    fr = jnp.zeros((Fp, N), f32).at[:F].set(jnp.real(d))
    fi = jnp.zeros((Fp, N), f32).at[:F].set(jnp.imag(d))
    fstack = jnp.concatenate([fr, fi], axis=0).astype(mm)
    arh_np, aih_np = _idft_half_mats(N, F, Fp)
    arh = jnp.asarray(arh_np.astype(np.float32), dtype=mm)
    aih = jnp.asarray(aih_np.astype(np.float32), dtype=mm)
    NHP = arh_np.shape[0]
    revm = jnp.asarray(np.eye(N // 2, dtype=np.float32)[::-1], dtype=mm)

    gmat = jnp.full((C, C), 1.0 / C, f32)
    smat = jnp.ones((C, 1), f32)
    xf = x.astype(f32)
    row = lambda v: v.astype(f32).reshape(1, -1)
    ln1w2, ln1b2 = row(ln1_w), row(ln1_b)
    ln2w2, ln2b2 = row(ln2_w), row(ln2_b)
    cwr, cwi = row(cw[:, 0]), row(cw[:, 1])
    chr_, chi_ = row(cwh[:, 0]), row(cwh[:, 1])
    w1 = conv1_w.astype(f32).astype(mm)
    w2 = conv2_w.astype(f32).reshape(3 * C, H).astype(mm)
    w3 = conv3_w.astype(f32).astype(mm)
    b1, b2 = row(conv1_b), row(conv2_b)
    b3 = row(conv3_b)

    cparams = pltpu.CompilerParams(
        dimension_semantics=("parallel",),
        vmem_limit_bytes=64 * 1024 * 1024,
    )
    bat = lambda i: (i, 0, 0)
    c2 = lambda i: (0, 0)
    BT = 2

    xfft, energy3 = pl.pallas_call(
        _pass_a,
        grid=(B,),
        in_specs=[
            pl.BlockSpec((1, N, C), bat),
            pl.BlockSpec((C, C), c2),
            pl.BlockSpec((C, 1), c2),
            pl.BlockSpec((1, C), c2),
            pl.BlockSpec((1, C), c2),
            pl.BlockSpec((2 * Fp, N), c2),
        ],
        out_specs=(
            pl.BlockSpec((1, 2 * Fp, C), bat),
            pl.BlockSpec((1, Fp, 1), bat),
        ),
        out_shape=(
            jax.ShapeDtypeStruct((B, 2 * Fp, C), mm),
            jax.ShapeDtypeStruct((B, Fp, 1), f32),
        ),
        compiler_params=cparams,
    )(xf, gmat, smat, ln1w2, ln1b2, fstack)

    pad = Fp - F
    energy = energy3.reshape(B, Fp)
    med = jnp.sort(energy, axis=1)[:, (F - 1) // 2 + pad][:, None]
    ne = energy / (med + _MED_EPS)
    nf = B * F
    v = jnp.sort(ne.ravel())
    pos = threshold.reshape(()).astype(f32) * np.float32(nf - 1)
    low = jnp.clip(jnp.floor(pos), 0, nf - 1).astype(jnp.int32)
    thr = jax.lax.dynamic_index_in_dim(v, B * pad + low, keepdims=False)
    mask3 = (ne > thr).astype(f32).reshape(B, Fp, 1)

    out = pl.pallas_call(
        _pass_b,
        grid=(B // BT,),
        in_specs=[
            pl.BlockSpec((BT, N, C), bat),
            pl.BlockSpec((BT, 2 * Fp, C), bat),
            pl.BlockSpec((BT, Fp, 1), bat),
            pl.BlockSpec((1, C), c2),
            pl.BlockSpec((1, C), c2),
            pl.BlockSpec((1, C), c2),
            pl.BlockSpec((1, C), c2),
            pl.BlockSpec((NHP, Fp), c2),
            pl.BlockSpec((NHP, Fp), c2),
            pl.BlockSpec((N // 2, N // 2), c2),
            pl.BlockSpec((1, C), c2),
            pl.BlockSpec((1, C), c2),
            pl.BlockSpec((C, H), c2),
            pl.BlockSpec((1, H), c2),
            pl.BlockSpec((3 * C, H), c2),
            pl.BlockSpec((1, H), c2),
            pl.BlockSpec((H, C), c2),
            pl.BlockSpec((1, C), c2),
        ],
        out_specs=pl.BlockSpec((BT, N, C), bat),
        out_shape=jax.ShapeDtypeStruct((B, N, C), f32),
        compiler_params=cparams,
    )(xf, xfft, mask3, cwr, cwi, chr_, chi_,
      arh, aih, revm, ln2w2, ln2b2, w1, b1, w2, b2, w3, b3)

    return out.astype(orig_dtype)


# speedup vs baseline: 1.0820x; 1.0063x over previous
import numpy as np

import jax
import jax.numpy as jnp
from jax.experimental import pallas as pl
from jax.experimental.pallas import tpu as pltpu

_LN_EPS = 1e-5
_MED_EPS = 1e-6
_SQRT_2_OVER_PI = 0.7978845608028654


def _gelu(x):
    return 0.5 * x * (1.0 + jnp.tanh(_SQRT_2_OVER_PI * (x + 0.044715 * x * x * x)))


def _idft_half_mats(N, F, Fp):
    nh = N // 2 + 1
    nhp = ((nh + 7) // 8) * 8
    n = np.arange(nh)[None, :]
    f = np.arange(F)[:, None]
    ang = 2.0 * np.pi * f * n / N
    scale = 1.0 / np.sqrt(N)
    w = np.full((F,), 2.0)
    w[0] = 1.0
    if N % 2 == 0:
        w[-1] = 1.0
    ar = np.zeros((nhp, Fp), np.float64)
    ai = np.zeros((nhp, Fp), np.float64)
    ar[:nh, :F] = np.cos(ang).T * (w * scale)
    ai[:nh, :F] = -np.sin(ang).T * (w * scale)
    return ar, ai


def _pass_a(x_ref, g_ref, s_ref, ln1w_ref, ln1b_ref, fst_ref, xfft_ref, energy_ref):
    x = x_ref[0]
    mu = jnp.dot(x, g_ref[...], preferred_element_type=jnp.float32)
    xc = x - mu
    var = jnp.dot(xc * xc, g_ref[...], preferred_element_type=jnp.float32)
    xn = xc * jax.lax.rsqrt(var + _LN_EPS) * ln1w_ref[...] + ln1b_ref[...]
    X = jnp.dot(fst_ref[...], xn.astype(fst_ref.dtype),
                preferred_element_type=jnp.float32)
    xfft_ref[0] = X.astype(xfft_ref.dtype)
    p = X * X
    e2 = jnp.dot(p, s_ref[...], preferred_element_type=jnp.float32)
    fp = e2.shape[0] // 2
    energy_ref[0] = e2[:fp] + e2[fp:]


def _pass_b(xres_ref, xfft_ref, mask_ref, cwr_ref, cwi_ref, chr_ref, chi_ref,
            arh_ref, aih_ref, rev_ref, ln2w_ref, ln2b_ref,
            w1_ref, b1_ref, w2_ref, b2_ref, w3_ref, b3_ref, o_ref):
    for j in range(xfft_ref.shape[0]):
        X = xfft_ref[j]
        fp = X.shape[0] // 2
        Xr, Xi = X[:fp], X[fp:]

        mask = mask_ref[j]
        wr_eff = cwr_ref[...] + mask * chr_ref[...]
        wi_eff = cwi_ref[...] + mask * chi_ref[...]
        Wr = Xr * wr_eff - Xi * wi_eff
        Wi = Xr * wi_eff + Xi * wr_eff

        u = jnp.dot(arh_ref[...], Wr, preferred_element_type=jnp.float32)
        v = jnp.dot(aih_ref[...], Wi, preferred_element_type=jnp.float32)
        n = xres_ref.shape[1]
        nh = n // 2
        top = u[:nh] + v[:nh]
        d = (u[1:nh + 1] - v[1:nh + 1]).astype(rev_ref.dtype)
        bot = jnp.dot(rev_ref[...], d, preferred_element_type=jnp.float32)
        xa = jnp.concatenate([top, bot], axis=0)

        mu = jnp.mean(xa, axis=1, keepdims=True)
        xc = xa - mu
        var = jnp.mean(xc * xc, axis=1, keepdims=True)
        y = xc * jax.lax.rsqrt(var + _LN_EPS) * ln2w_ref[...] + ln2b_ref[...]

        mm = w1_ref.dtype
        y_m = y.astype(mm)
        z = jnp.zeros((1, y.shape[1]), mm)
        y_prev = jnp.concatenate([z, y_m[:n - 1]], axis=0)
        y_next = jnp.concatenate([y_m[1:], z], axis=0)

        x1 = jnp.dot(y_m, w1_ref[...], preferred_element_type=jnp.float32) + b1_ref[...]
        taps = jnp.concatenate([y_prev, y_m, y_next], axis=1)
        x2 = jnp.dot(taps, w2_ref[...], preferred_element_type=jnp.float32) + b2_ref[...]
        xb1 = x1.astype(mm)
        xb2 = x2.astype(mm)
        a = jnp.asarray(0.044715, mm)
        c = jnp.asarray(_SQRT_2_OVER_PI, mm)
        t1 = jnp.tanh(c * (xb1 + a * xb1 * xb1 * xb1))
        t2 = jnp.tanh(c * (xb2 + a * xb2 * xb2 * xb2))
        s = jnp.asarray(1.0, mm) + jnp.asarray(0.5, mm) * (t1 + t2)
        out = xb1 * xb2 * s
        icb = jnp.dot(out, w3_ref[...],
                      preferred_element_type=jnp.float32) + b3_ref[...]
        o_ref[j] = xres_ref[j] + icb


@jax.jit
def kernel(x, cw, cwh, threshold, ln1_w, ln1_b, ln2_w, ln2_b,
           conv1_w, conv1_b, conv2_w, conv2_b, conv3_w, conv3_b):
    orig_dtype = x.dtype
    B, N, C = x.shape
    F = N // 2 + 1
    Fp = ((F + 7) // 8) * 8
    H = conv1_w.shape[1]
    f32 = jnp.float32
    mm = jnp.bfloat16

    d = jnp.fft.rfft(jnp.eye(N, dtype=f32), axis=0, norm="ortho")
    fr = jnp.zeros((Fp, N), f32).at[:F].set(jnp.real(d))
    fi = jnp.zeros((Fp, N), f32).at[:F].set(jnp.imag(d))
    fstack = jnp.concatenate([fr, fi], axis=0).astype(mm)
    arh_np, aih_np = _idft_half_mats(N, F, Fp)
    arh = jnp.asarray(arh_np.astype(np.float32), dtype=mm)
    aih = jnp.asarray(aih_np.astype(np.float32), dtype=mm)
    NHP = arh_np.shape[0]
    revm = jnp.asarray(np.eye(N // 2, dtype=np.float32)[::-1], dtype=mm)

    gmat = jnp.full((C, C), 1.0 / C, f32)
    smat = jnp.ones((C, 1), f32)
    xf = x.astype(f32)
    row = lambda v: v.astype(f32).reshape(1, -1)
    ln1w2, ln1b2 = row(ln1_w), row(ln1_b)
    ln2w2, ln2b2 = row(ln2_w), row(ln2_b)
    rowb = lambda v: v.astype(f32).reshape(1, -1).astype(mm)
    cwr, cwi = rowb(cw[:, 0]), rowb(cw[:, 1])
    chr_, chi_ = rowb(cwh[:, 0]), rowb(cwh[:, 1])
    w1 = conv1_w.astype(f32).astype(mm)
    w2 = conv2_w.astype(f32).reshape(3 * C, H).astype(mm)
    w3 = conv3_w.astype(f32).astype(mm)
    b1, b2 = row(conv1_b), row(conv2_b)
    b3 = row(conv3_b)

    cparams = pltpu.CompilerParams(
        dimension_semantics=("parallel",),
        vmem_limit_bytes=64 * 1024 * 1024,
    )
    bat = lambda i: (i, 0, 0)
    c2 = lambda i: (0, 0)
    BT = 2

    xfft, energy3 = pl.pallas_call(
        _pass_a,
        grid=(B,),
        in_specs=[
            pl.BlockSpec((1, N, C), bat),
            pl.BlockSpec((C, C), c2),
            pl.BlockSpec((C, 1), c2),
            pl.BlockSpec((1, C), c2),
            pl.BlockSpec((1, C), c2),
            pl.BlockSpec((2 * Fp, N), c2),
        ],
        out_specs=(
            pl.BlockSpec((1, 2 * Fp, C), bat),
            pl.BlockSpec((1, Fp, 1), bat),
        ),
        out_shape=(
            jax.ShapeDtypeStruct((B, 2 * Fp, C), mm),
            jax.ShapeDtypeStruct((B, Fp, 1), f32),
        ),
        compiler_params=cparams,
    )(xf, gmat, smat, ln1w2, ln1b2, fstack)

    pad = Fp - F
    energy = energy3.reshape(B, Fp)
    med = jnp.sort(energy, axis=1)[:, (F - 1) // 2 + pad][:, None]
    ne = energy / (med + _MED_EPS)
    nf = B * F
    v = jnp.sort(ne.ravel())
    pos = threshold.reshape(()).astype(f32) * np.float32(nf - 1)
    low = jnp.clip(jnp.floor(pos), 0, nf - 1).astype(jnp.int32)
    thr = jax.lax.dynamic_index_in_dim(v, B * pad + low, keepdims=False)
    mask3 = (ne > thr).astype(mm).reshape(B, Fp, 1)

    out = pl.pallas_call(
        _pass_b,
        grid=(B // BT,),
        in_specs=[
            pl.BlockSpec((BT, N, C), bat),
            pl.BlockSpec((BT, 2 * Fp, C), bat),
            pl.BlockSpec((BT, Fp, 1), bat),
            pl.BlockSpec((1, C), c2),
            pl.BlockSpec((1, C), c2),
            pl.BlockSpec((1, C), c2),
            pl.BlockSpec((1, C), c2),
            pl.BlockSpec((NHP, Fp), c2),
            pl.BlockSpec((NHP, Fp), c2),
            pl.BlockSpec((N // 2, N // 2), c2),
            pl.BlockSpec((1, C), c2),
            pl.BlockSpec((1, C), c2),
            pl.BlockSpec((C, H), c2),
            pl.BlockSpec((1, H), c2),
            pl.BlockSpec((3 * C, H), c2),
            pl.BlockSpec((1, H), c2),
            pl.BlockSpec((H, C), c2),
            pl.BlockSpec((1, C), c2),
        ],
        out_specs=pl.BlockSpec((BT, N, C), bat),
        out_shape=jax.ShapeDtypeStruct((B, N, C), f32),
        compiler_params=cparams,
    )(xf, xfft, mask3, cwr, cwi, chr_, chi_,
      arh, aih, revm, ln2w2, ln2b2, w1, b1, w2, b2, w3, b3)

    return out.astype(orig_dtype)


# speedup vs baseline: 1.0826x; 1.0006x over previous
import numpy as np

import jax
import jax.numpy as jnp
from jax.experimental import pallas as pl
from jax.experimental.pallas import tpu as pltpu

_LN_EPS = 1e-5
_MED_EPS = 1e-6
_SQRT_2_OVER_PI = 0.7978845608028654


def _gelu(x):
    return 0.5 * x * (1.0 + jnp.tanh(_SQRT_2_OVER_PI * (x + 0.044715 * x * x * x)))


_RFFT_CACHE = {}


def _rfft_mat_cached(N, F, Fp, dtype):
    key = (N, F, Fp, jnp.dtype(dtype).name)
    if key not in _RFFT_CACHE:
        d = jnp.fft.rfft(jnp.eye(N, dtype=jnp.float32), axis=0, norm="ortho")
        fr = jnp.zeros((Fp, N), jnp.float32).at[:F].set(jnp.real(d))
        fi = jnp.zeros((Fp, N), jnp.float32).at[:F].set(jnp.imag(d))
        _RFFT_CACHE[key] = jnp.concatenate([fr, fi], axis=0).astype(dtype)
    return _RFFT_CACHE[key]


def _idft_half_mats(N, F, Fp):
    nh = N // 2 + 1
    nhp = ((nh + 7) // 8) * 8
    n = np.arange(nh)[None, :]
    f = np.arange(F)[:, None]
    ang = 2.0 * np.pi * f * n / N
    scale = 1.0 / np.sqrt(N)
    w = np.full((F,), 2.0)
    w[0] = 1.0
    if N % 2 == 0:
        w[-1] = 1.0
    ar = np.zeros((nhp, Fp), np.float64)
    ai = np.zeros((nhp, Fp), np.float64)
    ar[:nh, :F] = np.cos(ang).T * (w * scale)
    ai[:nh, :F] = -np.sin(ang).T * (w * scale)
    return ar, ai


def _pass_a(x_ref, g_ref, s_ref, ln1w_ref, ln1b_ref, fst_ref, xfft_ref, energy_ref):
    x = x_ref[0]
    mu = jnp.dot(x, g_ref[...], preferred_element_type=jnp.float32)
    xc = x - mu
    var = jnp.dot(xc * xc, g_ref[...], preferred_element_type=jnp.float32)
    xn = xc * jax.lax.rsqrt(var + _LN_EPS) * ln1w_ref[...] + ln1b_ref[...]
    X = jnp.dot(fst_ref[...], xn.astype(fst_ref.dtype),
                preferred_element_type=jnp.float32)
    xfft_ref[0] = X.astype(xfft_ref.dtype)
    p = X * X
    e2 = jnp.dot(p, s_ref[...], preferred_element_type=jnp.float32)
    fp = e2.shape[0] // 2
    energy_ref[0] = e2[:fp] + e2[fp:]


def _pass_b(xres_ref, xfft_ref, mask_ref, cwr_ref, cwi_ref, chr_ref, chi_ref,
            arh_ref, aih_ref, rev_ref, ln2w_ref, ln2b_ref,
            w1_ref, b1_ref, w2_ref, b2_ref, w3_ref, b3_ref, o_ref):
    for j in range(xfft_ref.shape[0]):
        X = xfft_ref[j]
        fp = X.shape[0] // 2
        Xr, Xi = X[:fp], X[fp:]

        mask = mask_ref[j]
        wr_eff = cwr_ref[...] + mask * chr_ref[...]
        wi_eff = cwi_ref[...] + mask * chi_ref[...]
        Wr = Xr * wr_eff - Xi * wi_eff
        Wi = Xr * wi_eff + Xi * wr_eff

        u = jnp.dot(arh_ref[...], Wr, preferred_element_type=jnp.float32)
        v = jnp.dot(aih_ref[...], Wi, preferred_element_type=jnp.float32)
        n = xres_ref.shape[1]
        nh = n // 2
        top = u[:nh] + v[:nh]
        d = (u[1:nh + 1] - v[1:nh + 1]).astype(rev_ref.dtype)
        bot = jnp.dot(rev_ref[...], d, preferred_element_type=jnp.float32)
        xa = jnp.concatenate([top, bot], axis=0)

        mu = jnp.mean(xa, axis=1, keepdims=True)
        xc = xa - mu
        var = jnp.mean(xc * xc, axis=1, keepdims=True)
        y = xc * jax.lax.rsqrt(var + _LN_EPS) * ln2w_ref[...] + ln2b_ref[...]

        mm = w1_ref.dtype
        y_m = y.astype(mm)
        z = jnp.zeros((1, y.shape[1]), mm)
        y_prev = jnp.concatenate([z, y_m[:n - 1]], axis=0)
        y_next = jnp.concatenate([y_m[1:], z], axis=0)

        x1 = jnp.dot(y_m, w1_ref[...], preferred_element_type=jnp.float32) + b1_ref[...]
        taps = jnp.concatenate([y_prev, y_m, y_next], axis=1)
        x2 = jnp.dot(taps, w2_ref[...], preferred_element_type=jnp.float32) + b2_ref[...]
        xb1 = x1.astype(mm)
        xb2 = x2.astype(mm)
        a = jnp.asarray(0.044715, mm)
        c = jnp.asarray(_SQRT_2_OVER_PI, mm)
        t1 = jnp.tanh(c * (xb1 + a * xb1 * xb1 * xb1))
        t2 = jnp.tanh(c * (xb2 + a * xb2 * xb2 * xb2))
        s = jnp.asarray(1.0, mm) + jnp.asarray(0.5, mm) * (t1 + t2)
        out = xb1 * xb2 * s
        icb = jnp.dot(out, w3_ref[...],
                      preferred_element_type=jnp.float32) + b3_ref[...]
        o_ref[j] = xres_ref[j] + icb


@jax.jit
def kernel(x, cw, cwh, threshold, ln1_w, ln1_b, ln2_w, ln2_b,
           conv1_w, conv1_b, conv2_w, conv2_b, conv3_w, conv3_b):
    orig_dtype = x.dtype
    B, N, C = x.shape
    F = N // 2 + 1
    Fp = ((F + 7) // 8) * 8
    H = conv1_w.shape[1]
    f32 = jnp.float32
    mm = jnp.bfloat16

    fstack = _rfft_mat_cached(N, F, Fp, mm)
    arh_np, aih_np = _idft_half_mats(N, F, Fp)
    arh = jnp.asarray(arh_np.astype(np.float32), dtype=mm)
    aih = jnp.asarray(aih_np.astype(np.float32), dtype=mm)
    NHP = arh_np.shape[0]
    revm = jnp.asarray(np.eye(N // 2, dtype=np.float32)[::-1], dtype=mm)

    gmat = jnp.full((C, C), 1.0 / C, f32)
    smat = jnp.ones((C, 1), f32)
    xf = x.astype(f32)
    row = lambda v: v.astype(f32).reshape(1, -1)
    ln1w2, ln1b2 = row(ln1_w), row(ln1_b)
    ln2w2, ln2b2 = row(ln2_w), row(ln2_b)
    rowb = lambda v: v.astype(f32).reshape(1, -1).astype(mm)
    cwr, cwi = rowb(cw[:, 0]), rowb(cw[:, 1])
    chr_, chi_ = rowb(cwh[:, 0]), rowb(cwh[:, 1])
    w1 = conv1_w.astype(f32).astype(mm)
    w2 = conv2_w.astype(f32).reshape(3 * C, H).astype(mm)
    w3 = conv3_w.astype(f32).astype(mm)
    b1, b2 = row(conv1_b), row(conv2_b)
    b3 = row(conv3_b)

    cparams = pltpu.CompilerParams(
        dimension_semantics=("parallel",),
        vmem_limit_bytes=64 * 1024 * 1024,
    )
    bat = lambda i: (i, 0, 0)
    c2 = lambda i: (0, 0)
    BT = 2

    xfft, energy3 = pl.pallas_call(
        _pass_a,
        grid=(B,),
        in_specs=[
            pl.BlockSpec((1, N, C), bat),
            pl.BlockSpec((C, C), c2),
            pl.BlockSpec((C, 1), c2),
            pl.BlockSpec((1, C), c2),
            pl.BlockSpec((1, C), c2),
            pl.BlockSpec((2 * Fp, N), c2),
        ],
        out_specs=(
            pl.BlockSpec((1, 2 * Fp, C), bat),
            pl.BlockSpec((1, Fp, 1), bat),
        ),
        out_shape=(
            jax.ShapeDtypeStruct((B, 2 * Fp, C), mm),
            jax.ShapeDtypeStruct((B, Fp, 1), f32),
        ),
        compiler_params=cparams,
    )(xf, gmat, smat, ln1w2, ln1b2, fstack)

    pad = Fp - F
    energy = energy3.reshape(B, Fp)
    med = jnp.sort(energy, axis=1)[:, (F - 1) // 2 + pad][:, None]
    ne = energy / (med + _MED_EPS)
    nf = B * F
    v = jnp.sort(ne.ravel())
    pos = threshold.reshape(()).astype(f32) * np.float32(nf - 1)
    low = jnp.clip(jnp.floor(pos), 0, nf - 1).astype(jnp.int32)
    thr = jax.lax.dynamic_index_in_dim(v, B * pad + low, keepdims=False)
    mask3 = (ne > thr).astype(mm).reshape(B, Fp, 1)

    out = pl.pallas_call(
        _pass_b,
        grid=(B // BT,),
        in_specs=[
            pl.BlockSpec((BT, N, C), bat),
            pl.BlockSpec((BT, 2 * Fp, C), bat),
            pl.BlockSpec((BT, Fp, 1), bat),
            pl.BlockSpec((1, C), c2),
            pl.BlockSpec((1, C), c2),
            pl.BlockSpec((1, C), c2),
            pl.BlockSpec((1, C), c2),
            pl.BlockSpec((NHP, Fp), c2),
            pl.BlockSpec((NHP, Fp), c2),
            pl.BlockSpec((N // 2, N // 2), c2),
            pl.BlockSpec((1, C), c2),
            pl.BlockSpec((1, C), c2),
            pl.BlockSpec((C, H), c2),
            pl.BlockSpec((1, H), c2),
            pl.BlockSpec((3 * C, H), c2),
            pl.BlockSpec((1, H), c2),
            pl.BlockSpec((H, C), c2),
            pl.BlockSpec((1, C), c2),
        ],
        out_specs=pl.BlockSpec((BT, N, C), bat),
        out_shape=jax.ShapeDtypeStruct((B, N, C), f32),
        compiler_params=cparams,
    )(xf, xfft, mask3, cwr, cwi, chr_, chi_,
      arh, aih, revm, ln2w2, ln2b2, w1, b1, w2, b2, w3, b3)

    return out.astype(orig_dtype)


# speedup vs baseline: 1.1064x; 1.0220x over previous
import numpy as np

import jax
import jax.numpy as jnp
from jax.experimental import pallas as pl
from jax.experimental.pallas import tpu as pltpu

_LN_EPS = 1e-5
_MED_EPS = 1e-6
_SQRT_2_OVER_PI = 0.7978845608028654


def _gelu(x):
    return 0.5 * x * (1.0 + jnp.tanh(_SQRT_2_OVER_PI * (x + 0.044715 * x * x * x)))


_RFFT_CACHE = {}


def _rfft_mat_cached(N, F, Fp, dtype):
    key = (N, F, Fp, jnp.dtype(dtype).name)
    if key not in _RFFT_CACHE:
        d = jnp.fft.rfft(jnp.eye(N, dtype=jnp.float32), axis=0, norm="ortho")
        fr = jnp.zeros((Fp, N), jnp.float32).at[:F].set(jnp.real(d))
        fi = jnp.zeros((Fp, N), jnp.float32).at[:F].set(jnp.imag(d))
        _RFFT_CACHE[key] = jnp.concatenate([fr, fi], axis=0).astype(dtype)
    return _RFFT_CACHE[key]


def _idft_half_mats(N, F, Fp):
    nh = N // 2 + 1
    nhp = ((nh + 7) // 8) * 8
    n = np.arange(nh)[None, :]
    f = np.arange(F)[:, None]
    ang = 2.0 * np.pi * f * n / N
    scale = 1.0 / np.sqrt(N)
    w = np.full((F,), 2.0)
    w[0] = 1.0
    if N % 2 == 0:
        w[-1] = 1.0
    ar = np.zeros((nhp, Fp), np.float64)
    ai = np.zeros((nhp, Fp), np.float64)
    ar[:nh, :F] = np.cos(ang).T * (w * scale)
    ai[:nh, :F] = -np.sin(ang).T * (w * scale)
    return ar, ai


def _pass_a(x_ref, g_ref, s_ref, ln1w_ref, ln1b_ref, fst_ref, xfft_ref, energy_ref):
    x = x_ref[0]
    mu = jnp.dot(x, g_ref[...], preferred_element_type=jnp.float32)
    xc = x - mu
    var = jnp.dot(xc * xc, g_ref[...], preferred_element_type=jnp.float32)
    xn = xc * jax.lax.rsqrt(var + _LN_EPS) * ln1w_ref[...] + ln1b_ref[...]
    X = jnp.dot(fst_ref[...], xn.astype(fst_ref.dtype),
                preferred_element_type=jnp.float32)
    xfft_ref[0] = X.astype(xfft_ref.dtype)
    p = X * X
    e2 = jnp.dot(p, s_ref[...], preferred_element_type=jnp.float32)
    fp = e2.shape[0] // 2
    energy_ref[0] = jnp.transpose(e2[:fp] + e2[fp:])


def _pass_b(xres_ref, xfft_ref, mask_ref, cwr_ref, cwi_ref, chr_ref, chi_ref,
            arh_ref, aih_ref, rev_ref, ln2w_ref, ln2b_ref,
            w1_ref, b1_ref, w2_ref, b2_ref, w3_ref, b3_ref, o_ref):
    for j in range(xfft_ref.shape[0]):
        X = xfft_ref[j]
        fp = X.shape[0] // 2
        Xr, Xi = X[:fp], X[fp:]

        mask = jnp.transpose(mask_ref[j])
        wr_eff = cwr_ref[...] + mask * chr_ref[...]
        wi_eff = cwi_ref[...] + mask * chi_ref[...]
        Wr = Xr * wr_eff - Xi * wi_eff
        Wi = Xr * wi_eff + Xi * wr_eff

        u = jnp.dot(arh_ref[...], Wr, preferred_element_type=jnp.float32)
        v = jnp.dot(aih_ref[...], Wi, preferred_element_type=jnp.float32)
        n = xres_ref.shape[1]
        nh = n // 2
        top = u[:nh] + v[:nh]
        d = (u[1:nh + 1] - v[1:nh + 1]).astype(rev_ref.dtype)
        bot = jnp.dot(rev_ref[...], d, preferred_element_type=jnp.float32)
        xa = jnp.concatenate([top, bot], axis=0)

        mu = jnp.mean(xa, axis=1, keepdims=True)
        xc = xa - mu
        var = jnp.mean(xc * xc, axis=1, keepdims=True)
        y = xc * jax.lax.rsqrt(var + _LN_EPS) * ln2w_ref[...] + ln2b_ref[...]

        mm = w1_ref.dtype
        y_m = y.astype(mm)
        z = jnp.zeros((1, y.shape[1]), mm)
        y_prev = jnp.concatenate([z, y_m[:n - 1]], axis=0)
        y_next = jnp.concatenate([y_m[1:], z], axis=0)

        x1 = jnp.dot(y_m, w1_ref[...], preferred_element_type=jnp.float32) + b1_ref[...]
        taps = jnp.concatenate([y_prev, y_m, y_next], axis=1)
        x2 = jnp.dot(taps, w2_ref[...], preferred_element_type=jnp.float32) + b2_ref[...]
        xb1 = x1.astype(mm)
        xb2 = x2.astype(mm)
        a = jnp.asarray(0.044715, mm)
        c = jnp.asarray(_SQRT_2_OVER_PI, mm)
        t1 = jnp.tanh(c * (xb1 + a * xb1 * xb1 * xb1))
        t2 = jnp.tanh(c * (xb2 + a * xb2 * xb2 * xb2))
        s = jnp.asarray(1.0, mm) + jnp.asarray(0.5, mm) * (t1 + t2)
        out = xb1 * xb2 * s
        icb = jnp.dot(out, w3_ref[...],
                      preferred_element_type=jnp.float32) + b3_ref[...]
        o_ref[j] = xres_ref[j] + icb


@jax.jit
def kernel(x, cw, cwh, threshold, ln1_w, ln1_b, ln2_w, ln2_b,
           conv1_w, conv1_b, conv2_w, conv2_b, conv3_w, conv3_b):
    orig_dtype = x.dtype
    B, N, C = x.shape
    F = N // 2 + 1
    Fp = ((F + 7) // 8) * 8
    H = conv1_w.shape[1]
    f32 = jnp.float32
    mm = jnp.bfloat16

    fstack = _rfft_mat_cached(N, F, Fp, mm)
    arh_np, aih_np = _idft_half_mats(N, F, Fp)
    arh = jnp.asarray(arh_np.astype(np.float32), dtype=mm)
    aih = jnp.asarray(aih_np.astype(np.float32), dtype=mm)
    NHP = arh_np.shape[0]
    revm = jnp.asarray(np.eye(N // 2, dtype=np.float32)[::-1], dtype=mm)

    gmat = jnp.full((C, C), 1.0 / C, f32)
    smat = jnp.ones((C, 1), f32)
    xf = x.astype(f32)
    row = lambda v: v.astype(f32).reshape(1, -1)
    ln1w2, ln1b2 = row(ln1_w), row(ln1_b)
    ln2w2, ln2b2 = row(ln2_w), row(ln2_b)
    rowb = lambda v: v.astype(f32).reshape(1, -1).astype(mm)
    cwr, cwi = rowb(cw[:, 0]), rowb(cw[:, 1])
    chr_, chi_ = rowb(cwh[:, 0]), rowb(cwh[:, 1])
    w1 = conv1_w.astype(f32).astype(mm)
    w2 = conv2_w.astype(f32).reshape(3 * C, H).astype(mm)
    w3 = conv3_w.astype(f32).astype(mm)
    b1, b2 = row(conv1_b), row(conv2_b)
    b3 = row(conv3_b)

    cparams = pltpu.CompilerParams(
        dimension_semantics=("parallel",),
        vmem_limit_bytes=64 * 1024 * 1024,
    )
    bat = lambda i: (i, 0, 0)
    c2 = lambda i: (0, 0)
    BT = 2

    xfft, energy3 = pl.pallas_call(
        _pass_a,
        grid=(B,),
        in_specs=[
            pl.BlockSpec((1, N, C), bat),
            pl.BlockSpec((C, C), c2),
            pl.BlockSpec((C, 1), c2),
            pl.BlockSpec((1, C), c2),
            pl.BlockSpec((1, C), c2),
            pl.BlockSpec((2 * Fp, N), c2),
        ],
        out_specs=(
            pl.BlockSpec((1, 2 * Fp, C), bat),
            pl.BlockSpec((1, 1, Fp), bat),
        ),
        out_shape=(
            jax.ShapeDtypeStruct((B, 2 * Fp, C), mm),
            jax.ShapeDtypeStruct((B, 1, Fp), f32),
        ),
        compiler_params=cparams,
    )(xf, gmat, smat, ln1w2, ln1b2, fstack)

    pad = Fp - F
    energy = energy3.reshape(B, Fp)
    med = jnp.sort(energy, axis=1)[:, (F - 1) // 2 + pad][:, None]
    ne = energy / (med + _MED_EPS)
    nf = B * F
    v = jnp.sort(ne.ravel())
    pos = threshold.reshape(()).astype(f32) * np.float32(nf - 1)
    low = jnp.clip(jnp.floor(pos), 0, nf - 1).astype(jnp.int32)
    thr = jax.lax.dynamic_index_in_dim(v, B * pad + low, keepdims=False)
    mask3 = (ne > thr).astype(mm).reshape(B, 1, Fp)

    out = pl.pallas_call(
        _pass_b,
        grid=(B // BT,),
        in_specs=[
            pl.BlockSpec((BT, N, C), bat),
            pl.BlockSpec((BT, 2 * Fp, C), bat),
            pl.BlockSpec((BT, 1, Fp), bat),
            pl.BlockSpec((1, C), c2),
            pl.BlockSpec((1, C), c2),
            pl.BlockSpec((1, C), c2),
            pl.BlockSpec((1, C), c2),
            pl.BlockSpec((NHP, Fp), c2),
            pl.BlockSpec((NHP, Fp), c2),
            pl.BlockSpec((N // 2, N // 2), c2),
            pl.BlockSpec((1, C), c2),
            pl.BlockSpec((1, C), c2),
            pl.BlockSpec((C, H), c2),
            pl.BlockSpec((1, H), c2),
            pl.BlockSpec((3 * C, H), c2),
            pl.BlockSpec((1, H), c2),
            pl.BlockSpec((H, C), c2),
            pl.BlockSpec((1, C), c2),
        ],
        out_specs=pl.BlockSpec((BT, N, C), bat),
        out_shape=jax.ShapeDtypeStruct((B, N, C), f32),
        compiler_params=cparams,
    )(xf, xfft, mask3, cwr, cwi, chr_, chi_,
      arh, aih, revm, ln2w2, ln2b2, w1, b1, w2, b2, w3, b3)

    return out.astype(orig_dtype)


# speedup vs baseline: 1.2644x; 1.1428x over previous
import numpy as np

import jax
import jax.numpy as jnp
from jax.experimental import pallas as pl
from jax.experimental.pallas import tpu as pltpu

_LN_EPS = 1e-5
_MED_EPS = 1e-6
_SQRT_2_OVER_PI = 0.7978845608028654


def _gelu(x):
    return 0.5 * x * (1.0 + jnp.tanh(_SQRT_2_OVER_PI * (x + 0.044715 * x * x * x)))


def _dft_mat(N, F, Fp):
    n = np.arange(N)[None, :]
    f = np.arange(F)[:, None]
    ang = 2.0 * np.pi * f * n / N
    scale = 1.0 / np.sqrt(N)
    fr = np.zeros((Fp, N), np.float64)
    fi = np.zeros((Fp, N), np.float64)
    fr[:F] = np.cos(ang) * scale
    fi[:F] = -np.sin(ang) * scale
    return np.concatenate([fr, fi], axis=0)


def _idft_half_mats(N, F, Fp):
    nh = N // 2 + 1
    nhp = ((nh + 7) // 8) * 8
    n = np.arange(nh)[None, :]
    f = np.arange(F)[:, None]
    ang = 2.0 * np.pi * f * n / N
    scale = 1.0 / np.sqrt(N)
    w = np.full((F,), 2.0)
    w[0] = 1.0
    if N % 2 == 0:
        w[-1] = 1.0
    ar = np.zeros((nhp, Fp), np.float64)
    ai = np.zeros((nhp, Fp), np.float64)
    ar[:nh, :F] = np.cos(ang).T * (w * scale)
    ai[:nh, :F] = -np.sin(ang).T * (w * scale)
    return ar, ai


def _pass_a(x_ref, g_ref, s_ref, ln1w_ref, ln1b_ref, fst_ref, xfft_ref, energy_ref):
    x = x_ref[0]
    mu = jnp.dot(x, g_ref[...], preferred_element_type=jnp.float32)
    xc = x - mu
    var = jnp.dot(xc * xc, g_ref[...], preferred_element_type=jnp.float32)
    xn = xc * jax.lax.rsqrt(var + _LN_EPS) * ln1w_ref[...] + ln1b_ref[...]
    X = jnp.dot(fst_ref[...], xn.astype(fst_ref.dtype),
                preferred_element_type=jnp.float32)
    xfft_ref[0] = X.astype(xfft_ref.dtype)
    p = X * X
    e2 = jnp.dot(p, s_ref[...], preferred_element_type=jnp.float32)
    fp = e2.shape[0] // 2
    energy_ref[0] = jnp.transpose(e2[:fp] + e2[fp:])


def _pass_b(xres_ref, xfft_ref, mask_ref, cwr_ref, cwi_ref, chr_ref, chi_ref,
            arh_ref, aih_ref, rev_ref, ln2w_ref, ln2b_ref,
            w1_ref, b1_ref, w2_ref, b2_ref, w3_ref, b3_ref, o_ref):
    for j in range(xfft_ref.shape[0]):
        X = xfft_ref[j]
        fp = X.shape[0] // 2
        Xr, Xi = X[:fp], X[fp:]

        mask = jnp.transpose(mask_ref[j])
        wr_eff = cwr_ref[...] + mask * chr_ref[...]
        wi_eff = cwi_ref[...] + mask * chi_ref[...]
        Wr = Xr * wr_eff - Xi * wi_eff
        Wi = Xr * wi_eff + Xi * wr_eff

        u = jnp.dot(arh_ref[...], Wr, preferred_element_type=jnp.float32)
        v = jnp.dot(aih_ref[...], Wi, preferred_element_type=jnp.float32)
        n = xres_ref.shape[1]
        nh = n // 2
        top = u[:nh] + v[:nh]
        d = (u[1:nh + 1] - v[1:nh + 1]).astype(rev_ref.dtype)
        bot = jnp.dot(rev_ref[...], d, preferred_element_type=jnp.float32)
        xa = jnp.concatenate([top, bot], axis=0)

        mu = jnp.mean(xa, axis=1, keepdims=True)
        xc = xa - mu
        var = jnp.mean(xc * xc, axis=1, keepdims=True)
        y = xc * jax.lax.rsqrt(var + _LN_EPS) * ln2w_ref[...] + ln2b_ref[...]

        mm = w1_ref.dtype
        y_m = y.astype(mm)
        z = jnp.zeros((1, y.shape[1]), mm)
        y_prev = jnp.concatenate([z, y_m[:n - 1]], axis=0)
        y_next = jnp.concatenate([y_m[1:], z], axis=0)

        x1 = jnp.dot(y_m, w1_ref[...], preferred_element_type=jnp.float32) + b1_ref[...]
        taps = jnp.concatenate([y_prev, y_m, y_next], axis=1)
        x2 = jnp.dot(taps, w2_ref[...], preferred_element_type=jnp.float32) + b2_ref[...]
        xb1 = x1.astype(mm)
        xb2 = x2.astype(mm)
        a = jnp.asarray(0.044715, mm)
        c = jnp.asarray(_SQRT_2_OVER_PI, mm)
        t1 = jnp.tanh(c * (xb1 + a * xb1 * xb1 * xb1))
        t2 = jnp.tanh(c * (xb2 + a * xb2 * xb2 * xb2))
        s = jnp.asarray(1.0, mm) + jnp.asarray(0.5, mm) * (t1 + t2)
        out = xb1 * xb2 * s
        icb = jnp.dot(out, w3_ref[...],
                      preferred_element_type=jnp.float32) + b3_ref[...]
        o_ref[j] = xres_ref[j] + icb


@jax.jit
def kernel(x, cw, cwh, threshold, ln1_w, ln1_b, ln2_w, ln2_b,
           conv1_w, conv1_b, conv2_w, conv2_b, conv3_w, conv3_b):
    orig_dtype = x.dtype
    B, N, C = x.shape
    F = N // 2 + 1
    Fp = ((F + 7) // 8) * 8
    H = conv1_w.shape[1]
    f32 = jnp.float32
    mm = jnp.bfloat16

    fstack = jnp.asarray(_dft_mat(N, F, Fp).astype(np.float32), dtype=mm)
    arh_np, aih_np = _idft_half_mats(N, F, Fp)
    arh = jnp.asarray(arh_np.astype(np.float32), dtype=mm)
    aih = jnp.asarray(aih_np.astype(np.float32), dtype=mm)
    NHP = arh_np.shape[0]
    revm = jnp.asarray(np.eye(N // 2, dtype=np.float32)[::-1], dtype=mm)

    gmat = jnp.full((C, C), 1.0 / C, f32)
    smat = jnp.ones((C, 1), f32)
    xf = x.astype(f32)
    row = lambda v: v.astype(f32).reshape(1, -1)
    ln1w2, ln1b2 = row(ln1_w), row(ln1_b)
    ln2w2, ln2b2 = row(ln2_w), row(ln2_b)
    rowb = lambda v: v.astype(f32).reshape(1, -1).astype(mm)
    cwr, cwi = rowb(cw[:, 0]), rowb(cw[:, 1])
    chr_, chi_ = rowb(cwh[:, 0]), rowb(cwh[:, 1])
    w1 = conv1_w.astype(f32).astype(mm)
    w2 = conv2_w.astype(f32).reshape(3 * C, H).astype(mm)
    w3 = conv3_w.astype(f32).astype(mm)
    b1, b2 = row(conv1_b), row(conv2_b)
    b3 = row(conv3_b)

    cparams = pltpu.CompilerParams(
        dimension_semantics=("parallel",),
        vmem_limit_bytes=64 * 1024 * 1024,
    )
    bat = lambda i: (i, 0, 0)
    c2 = lambda i: (0, 0)
    BT = 2

    xfft, energy3 = pl.pallas_call(
        _pass_a,
        grid=(B,),
        in_specs=[
            pl.BlockSpec((1, N, C), bat),
            pl.BlockSpec((C, C), c2),
            pl.BlockSpec((C, 1), c2),
            pl.BlockSpec((1, C), c2),
            pl.BlockSpec((1, C), c2),
            pl.BlockSpec((2 * Fp, N), c2),
        ],
        out_specs=(
            pl.BlockSpec((1, 2 * Fp, C), bat),
            pl.BlockSpec((1, 1, Fp), bat),
        ),
        out_shape=(
            jax.ShapeDtypeStruct((B, 2 * Fp, C), mm),
            jax.ShapeDtypeStruct((B, 1, Fp), f32),
        ),
        compiler_params=cparams,
    )(xf, gmat, smat, ln1w2, ln1b2, fstack)

    pad = Fp - F
    energy = energy3.reshape(B, Fp)
    med = jnp.sort(energy, axis=1)[:, (F - 1) // 2 + pad][:, None]
    ne = energy / (med + _MED_EPS)
    nf = B * F
    v = jnp.sort(ne.ravel())
    pos = threshold.reshape(()).astype(f32) * np.float32(nf - 1)
    low = jnp.clip(jnp.floor(pos), 0, nf - 1).astype(jnp.int32)
    thr = jax.lax.dynamic_index_in_dim(v, B * pad + low, keepdims=False)
    mask3 = (ne > thr).astype(mm).reshape(B, 1, Fp)

    out = pl.pallas_call(
        _pass_b,
        grid=(B // BT,),
        in_specs=[
            pl.BlockSpec((BT, N, C), bat),
            pl.BlockSpec((BT, 2 * Fp, C), bat),
            pl.BlockSpec((BT, 1, Fp), bat),
            pl.BlockSpec((1, C), c2),
            pl.BlockSpec((1, C), c2),
            pl.BlockSpec((1, C), c2),
            pl.BlockSpec((1, C), c2),
            pl.BlockSpec((NHP, Fp), c2),
            pl.BlockSpec((NHP, Fp), c2),
            pl.BlockSpec((N // 2, N // 2), c2),
            pl.BlockSpec((1, C), c2),
            pl.BlockSpec((1, C), c2),
            pl.BlockSpec((C, H), c2),
            pl.BlockSpec((1, H), c2),
            pl.BlockSpec((3 * C, H), c2),
            pl.BlockSpec((1, H), c2),
            pl.BlockSpec((H, C), c2),
            pl.BlockSpec((1, C), c2),
        ],
        out_specs=pl.BlockSpec((BT, N, C), bat),
        out_shape=jax.ShapeDtypeStruct((B, N, C), f32),
        compiler_params=cparams,
    )(xf, xfft, mask3, cwr, cwi, chr_, chi_,
      arh, aih, revm, ln2w2, ln2b2, w1, b1, w2, b2, w3, b3)

    return out.astype(orig_dtype)
